# baseline (speedup 1.0000x reference)
import functools

import jax
import jax.numpy as jnp
from jax import lax
from jax.experimental import pallas as pl
from jax.experimental.pallas import tpu as pltpu
from jax.experimental.pallas import tpu_sc as plsc

_B = 4
_T = 2048
_D = 4096
_E = 64
_K = 8
_EPS = 1e-6
_TOK = 512

_BT = _B * _T
_LANES = 16
_WORKERS = 32
_CHUNK = _BT // _WORKERS
_GROUPS = _CHUNK // _LANES
_NEG_INF = float("-inf")


def _router_body(hs_hbm, topv_hbm, topi_hbm, x_v, tv_v, ti_v):
    wid = lax.axis_index("s") * 2 + lax.axis_index("c")
    pltpu.sync_copy(hs_hbm.at[pl.ds(wid * _CHUNK * _E, _CHUNK * _E)], x_v)
    lanes = lax.iota(jnp.int32, _LANES)

    def group(g, carry):
        row = (g * _LANES + lanes) * _E
        out_row = (g * _LANES + lanes) * _K
        for k in range(_K):
            m = jnp.full((_LANES,), _NEG_INF, jnp.float32)
            a = jnp.zeros((_LANES,), jnp.int32)
            for e in range(_E):
                xe = plsc.load_gather(x_v, [row + e])
                gt = xe > m
                m = jnp.where(gt, xe, m)
                a = jnp.where(gt, jnp.full((_LANES,), e, jnp.int32), a)
            plsc.store_scatter(x_v, [row + a], jnp.full((_LANES,), _NEG_INF, jnp.float32))
            plsc.store_scatter(tv_v, [out_row + k], m)
            plsc.store_scatter(ti_v, [out_row + k], a)
        return carry

    lax.fori_loop(0, _GROUPS, group, 0)
    pltpu.sync_copy(tv_v, topv_hbm.at[pl.ds(wid * _CHUNK * _K, _CHUNK * _K)])
    pltpu.sync_copy(ti_v, topi_hbm.at[pl.ds(wid * _CHUNK * _K, _CHUNK * _K)])


_router = functools.partial(
    pl.kernel,
    mesh=plsc.VectorSubcoreMesh(core_axis_name="c", subcore_axis_name="s"),
    compiler_params=pltpu.CompilerParams(needs_layout_passes=False),
    out_type=[
        jax.ShapeDtypeStruct((_BT * _K,), jnp.float32),
        jax.ShapeDtypeStruct((_BT * _K,), jnp.int32),
    ],
    scratch_types=[
        pltpu.VMEM((_CHUNK * _E,), jnp.float32),
        pltpu.VMEM((_CHUNK * _K,), jnp.float32),
        pltpu.VMEM((_CHUNK * _K,), jnp.int32),
    ],
)(_router_body)


def _mix_body(hs_full, topv, topi, limes, out_ref, scale_ref):
    i = pl.program_id(0)

    @pl.when(i == 0)
    def _():
        scale_ref[0, 0] = jnp.maximum(jnp.max(jnp.abs(hs_full[...])), _EPS)

    inv_s = 1.0 / scale_ref[0, 0]
    v = topv[...]
    e = jnp.exp((v - v[:, 0:1]) * inv_s)
    w = e / jnp.sum(e, axis=-1, keepdims=True)

    iota = jax.lax.broadcasted_iota(jnp.int32, (_TOK, _E), 1).astype(jnp.float32)
    ti = topi[...].astype(jnp.float32)
    dense_w = jnp.zeros((_TOK, _E), jnp.float32)
    for k in range(_K):
        dense_w = dense_w + jnp.where(iota == ti[:, k : k + 1], w[:, k : k + 1], 0.0)

    out_ref[...] = jnp.dot(dense_w, limes[...], preferred_element_type=jnp.float32)


def kernel(H, LiMEs):
    H2 = H.reshape(_BT, _D)
    hs = H2[:, :_E]
    topv_flat, topi_flat = _router(hs.reshape(_BT * _E))
    topv = topv_flat.reshape(_BT, _K)
    topi = topi_flat.reshape(_BT, _K)
    out = pl.pallas_call(
        _mix_body,
        grid=(_BT // _TOK,),
        in_specs=[
            pl.BlockSpec((_BT, _E), lambda i: (0, 0)),
            pl.BlockSpec((_TOK, _K), lambda i: (i, 0)),
            pl.BlockSpec((_TOK, _K), lambda i: (i, 0)),
            pl.BlockSpec((_E, _D), lambda i: (0, 0)),
        ],
        out_specs=pl.BlockSpec((_TOK, _D), lambda i: (i, 0)),
        out_shape=jax.ShapeDtypeStruct((_BT, _D), jnp.float32),
        scratch_shapes=[pltpu.SMEM((1, 1), jnp.float32)],
    )(hs, topv, topi, LiMEs)
    p_mix = out.reshape(_B, _T, _D)
    topk_idx = topi.reshape(_B, _T, _K)
    return p_mix, topk_idx


# speedup vs baseline: 1.5039x; 1.5039x over previous
import functools

import jax
import jax.numpy as jnp
from jax import lax
from jax.experimental import pallas as pl
from jax.experimental.pallas import tpu as pltpu
from jax.experimental.pallas import tpu_sc as plsc

_B = 4
_T = 2048
_D = 4096
_E = 64
_K = 8
_EPS = 1e-6
_TOK = 512

_BT = _B * _T
_LANES = 16
_WORKERS = 32
_CHUNK = _BT // _WORKERS
_GROUPS = _CHUNK // _LANES
_NEG_INF = float("-inf")


def _router_body(hs_hbm, topv_hbm, topi_hbm, x_v, tv_v, ti_v):
    wid = lax.axis_index("s") * 2 + lax.axis_index("c")
    pltpu.sync_copy(hs_hbm.at[pl.ds(wid * _CHUNK * _E, _CHUNK * _E)], x_v)
    lanes = lax.iota(jnp.int32, _LANES)

    def group(g, carry):
        toff = g * _LANES
        tok = toff + lanes
        for k in range(_K):
            m = jnp.full((_LANES,), _NEG_INF, jnp.float32)
            a = jnp.zeros((_LANES,), jnp.int32)
            for e in range(_E):
                xe = x_v[pl.ds(e * _CHUNK + toff, _LANES)]
                gt = xe > m
                m = jnp.where(gt, xe, m)
                a = jnp.where(gt, jnp.full((_LANES,), e, jnp.int32), a)
            plsc.store_scatter(
                x_v, [a * _CHUNK + tok], jnp.full((_LANES,), _NEG_INF, jnp.float32)
            )
            tv_v[pl.ds(k * _CHUNK + toff, _LANES)] = m
            ti_v[pl.ds(k * _CHUNK + toff, _LANES)] = a
        return carry

    lax.fori_loop(0, _GROUPS, group, 0)
    pltpu.sync_copy(tv_v, topv_hbm.at[pl.ds(wid * _CHUNK * _K, _CHUNK * _K)])
    pltpu.sync_copy(ti_v, topi_hbm.at[pl.ds(wid * _CHUNK * _K, _CHUNK * _K)])


_router = functools.partial(
    pl.kernel,
    mesh=plsc.VectorSubcoreMesh(core_axis_name="c", subcore_axis_name="s"),
    compiler_params=pltpu.CompilerParams(needs_layout_passes=False),
    out_type=[
        jax.ShapeDtypeStruct((_WORKERS * _K * _CHUNK,), jnp.float32),
        jax.ShapeDtypeStruct((_WORKERS * _K * _CHUNK,), jnp.int32),
    ],
    scratch_types=[
        pltpu.VMEM((_E * _CHUNK,), jnp.float32),
        pltpu.VMEM((_K * _CHUNK,), jnp.float32),
        pltpu.VMEM((_K * _CHUNK,), jnp.int32),
    ],
)(_router_body)


def _mix_body(hs_full, topv, topi, limes, out_ref, scale_ref):
    i = pl.program_id(0)

    @pl.when(i == 0)
    def _():
        scale_ref[0, 0] = jnp.maximum(jnp.max(jnp.abs(hs_full[...])), _EPS)

    inv_s = 1.0 / scale_ref[0, 0]
    v = topv[...]
    e = jnp.exp((v - v[:, 0:1]) * inv_s)
    w = e / jnp.sum(e, axis=-1, keepdims=True)

    iota = jax.lax.broadcasted_iota(jnp.int32, (_TOK, _E), 1).astype(jnp.float32)
    tif = topi[...].astype(jnp.float32)
    dense_w = jnp.zeros((_TOK, _E), jnp.float32)
    for k in range(_K):
        dense_w = dense_w + jnp.where(iota == tif[:, k : k + 1], w[:, k : k + 1], 0.0)

    out_ref[...] = jnp.dot(dense_w, limes[...], preferred_element_type=jnp.float32)


def kernel(H, LiMEs):
    H2 = H.reshape(_BT, _D)
    hs_prep = (
        H2[:, :_E].T.reshape(_E, _WORKERS, _CHUNK).transpose(1, 0, 2).reshape(-1)
    )
    topv_flat, topi_flat = _router(hs_prep)
    topv = topv_flat.reshape(_WORKERS, _K, _CHUNK).transpose(0, 2, 1).reshape(_BT, _K)
    topi = topi_flat.reshape(_WORKERS, _K, _CHUNK).transpose(0, 2, 1).reshape(_BT, _K)
    out = pl.pallas_call(
        _mix_body,
        grid=(_BT // _TOK,),
        in_specs=[
            pl.BlockSpec((_BT // 4, _E * 4), lambda i: (0, 0)),
            pl.BlockSpec((_TOK, _K), lambda i: (i, 0)),
            pl.BlockSpec((_TOK, _K), lambda i: (i, 0)),
            pl.BlockSpec((_E, _D), lambda i: (0, 0)),
        ],
        out_specs=pl.BlockSpec((_TOK, _D), lambda i: (i, 0)),
        out_shape=jax.ShapeDtypeStruct((_BT, _D), jnp.float32),
        scratch_shapes=[pltpu.SMEM((1, 1), jnp.float32)],
    )(hs_prep.reshape(_BT // 4, _E * 4), topv, topi, LiMEs)
    p_mix = out.reshape(_B, _T, _D)
    topk_idx = topi.reshape(_B, _T, _K)
    return p_mix, topk_idx
